# baseline (speedup 1.0000x reference)
import functools

import jax
import jax.numpy as jnp
from jax import lax
from jax.experimental import pallas as pl
from jax.experimental.pallas import tpu as pltpu
from jax.experimental.pallas import tpu_sc as plsc

N_NODES = 10000
N_EDGES = 320000
D = 128
LANES = 16

NUM_CORES = 2
NUM_SUBCORES = 16
NUM_WORKERS = NUM_CORES * NUM_SUBCORES
EDGES_PER_WORKER = N_EDGES // NUM_WORKERS
BATCH = 80
NB = EDGES_PER_WORKER // BATCH
NBP = 128
NBW = NBP + 3
EROW = 160
N_PAD = 10240
ROWS_PER_TILE = N_PAD // NUM_SUBCORES
GROUPS = BATCH // LANES
BLOCKS = D // LANES


def _sc_segment_sum(features, edata, evals):
  mesh = plsc.VectorSubcoreMesh(core_axis_name="c", subcore_axis_name="s")

  @functools.partial(
      pl.kernel,
      mesh=mesh,
      out_type=jax.ShapeDtypeStruct((NUM_CORES, N_PAD, D), jnp.float32),
      scratch_types=[
          [pltpu.VMEM((EROW,), jnp.int32) for _ in range(3)],
          [pltpu.VMEM((BATCH,), jnp.float32) for _ in range(3)],
          [pltpu.VMEM((BATCH, D), jnp.float32) for _ in range(3)],
          pltpu.VMEM_SHARED((N_PAD, D), jnp.float32),
          [pltpu.SemaphoreType.DMA for _ in range(3)],
          [pltpu.SemaphoreType.DMA for _ in range(3)],
          [pltpu.SemaphoreType.DMA for _ in range(3)],
      ],
  )
  def k(feat_hbm, edata_hbm, eval_hbm, out_hbm, ebufs, vbufs, rows_bufs,
        accum, lsems, gsems, csems):
    c = lax.axis_index("c")
    s = lax.axis_index("s")
    wid = s * NUM_CORES + c
    row_base = wid * NBW

    def issue_l(r, b):
      pltpu.async_copy(edata_hbm.at[row_base + r], ebufs[b], lsems[b])
      pltpu.async_copy(eval_hbm.at[row_base + r], vbufs[b], lsems[b])

    def wait_l(r, b):
      pltpu.make_async_copy(edata_hbm.at[row_base + r], ebufs[b],
                            lsems[b]).wait()
      pltpu.make_async_copy(eval_hbm.at[row_base + r], vbufs[b],
                            lsems[b]).wait()

    def issue_g(b):
      pltpu.async_copy(feat_hbm.at[ebufs[b].at[pl.ds(0, BATCH)]],
                       rows_bufs[b], gsems[b])

    def wait_g(b):
      pltpu.make_async_copy(feat_hbm.at[ebufs[b].at[pl.ds(0, BATCH)]],
                            rows_bufs[b], gsems[b]).wait()

    def scale(b):
      def body(g, carry):
        vv = vbufs[b][pl.ds(g * LANES, LANES)]
        for lane in range(LANES):
          v = vv[lane]
          e = g * LANES + lane
          for j in range(BLOCKS):
            sl = pl.ds(j * LANES, LANES)
            rows_bufs[b][e, sl] = rows_bufs[b][e, sl] * v
        return carry

      lax.fori_loop(0, GROUPS, body, None)

    def issue_c(b):
      for i in range(GROUPS):
        dvec = ebufs[b][pl.ds(BATCH + i * LANES, LANES)]
        pltpu.async_copy(rows_bufs[b].at[pl.ds(i * LANES, LANES)],
                         accum.at[dvec], csems[b], add=True)

    def wait_c(b):
      for i in range(GROUPS):
        dvec = ebufs[b][pl.ds(BATCH + i * LANES, LANES)]
        pltpu.make_async_copy(rows_bufs[b].at[pl.ds(i * LANES, LANES)],
                              accum.at[dvec], csems[b]).wait()

    issue_l(0, 0)
    issue_l(1, 1)
    issue_l(2, 2)

    zeros = jnp.zeros((LANES,), jnp.float32)

    def zero_body(i, carry):
      for j in range(BLOCKS):
        rows_bufs[0][i, pl.ds(j * LANES, LANES)] = zeros
      return carry

    lax.fori_loop(0, BATCH, zero_body, None)
    for kk in range(ROWS_PER_TILE // BATCH):
      pltpu.sync_copy(
          rows_bufs[0].at[pl.ds(0, BATCH)],
          accum.at[pl.ds(s * ROWS_PER_TILE + kk * BATCH, BATCH)])
    plsc.subcore_barrier()

    wait_l(0, 0)
    issue_g(0)
    for r in (0, 1):
      x, y = r % 3, (r + 1) % 3
      wait_l(r + 1, y)
      issue_g(y)
      wait_g(x)
      scale(x)
      issue_c(x)
      issue_l(r + 3, x)

    def steady(kk, carry):
      r0 = 3 * kk + 2
      for rr in range(3):
        r = r0 + rr
        x, y = (2 + rr) % 3, (rr) % 3
        wait_l(r + 1, y)
        wait_c(y)
        issue_g(y)
        wait_g(x)
        scale(x)
        issue_c(x)
        issue_l(r + 3, x)
      return carry

    lax.fori_loop(0, (NBP - 2) // 3, steady, None)

    wait_l(NBP + 1, 0)
    wait_l(NBP + 2, 1)
    wait_g(2)
    wait_c(0)
    wait_c(1)

    plsc.subcore_barrier()
    base = s * ROWS_PER_TILE
    pltpu.sync_copy(accum.at[pl.ds(base, ROWS_PER_TILE)],
                    out_hbm.at[c, pl.ds(base, ROWS_PER_TILE)])

  return k(features, edata, evals)


def _tc_combine_matmul(p0, p1, w):
  block_rows = 1000

  def body(p0_ref, p1_ref, w_ref, out_ref):
    out_ref[...] = jnp.dot(p0_ref[...] + p1_ref[...], w_ref[...],
                           preferred_element_type=jnp.float32)

  return pl.pallas_call(
      body,
      grid=(N_NODES // block_rows,),
      in_specs=[
          pl.BlockSpec((block_rows, D), lambda i: (i, 0)),
          pl.BlockSpec((block_rows, D), lambda i: (i, 0)),
          pl.BlockSpec((D, D), lambda i: (0, 0)),
      ],
      out_specs=pl.BlockSpec((block_rows, D), lambda i: (i, 0)),
      out_shape=jax.ShapeDtypeStruct((N_NODES, D), jnp.float32),
  )(p0, p1, w)


def _pack_edata(edge_index, edge_values):
  pad_rows = NBW - NB
  src = edge_index[0].reshape(NUM_WORKERS, NB, BATCH)
  dst = edge_index[1].reshape(NUM_WORKERS, NB, BATCH)
  vals = edge_values.reshape(NUM_WORKERS, NB, BATCH)
  zpad = jnp.zeros((NUM_WORKERS, pad_rows, BATCH), jnp.int32)
  src = jnp.concatenate([src, zpad], axis=1)
  dst = jnp.concatenate([dst, zpad], axis=1)
  vals = jnp.concatenate([vals, zpad.astype(jnp.float32)], axis=1)
  edata = jnp.concatenate([src, dst], axis=2).reshape(NUM_WORKERS * NBW, EROW)
  evals = vals.reshape(NUM_WORKERS * NBW, BATCH)
  return edata, evals


def kernel(features, edge_index, edge_values, W):
  edata, evals = _pack_edata(edge_index, edge_values)
  partials = _sc_segment_sum(features, edata, evals)
  return _tc_combine_matmul(partials[0], partials[1], W)
